# baseline (speedup 1.0000x reference)
import functools

import jax
import jax.numpy as jnp
from jax import lax
from jax.experimental import pallas as pl
from jax.experimental.pallas import tpu as pltpu
from jax.experimental.pallas import tpu_sc as plsc

V = 100000
NNEG = 5
B = 4096
NC, NS, L = 2, 16, 16
NW = NC * NS
RPT = B // NW
SLOTS = 1 + NNEG

_mesh = plsc.VectorSubcoreMesh(core_axis_name="c", subcore_axis_name="s")


@functools.partial(
    pl.kernel,
    mesh=_mesh,
    out_type=jax.ShapeDtypeStruct((NW, SLOTS, RPT), jnp.float32),
    scratch_types=[
        pltpu.VMEM((RPT,), jnp.int32),
        pltpu.VMEM((NNEG * RPT,), jnp.int32),
        pltpu.VMEM((SLOTS, RPT), jnp.int32),
        pltpu.VMEM((SLOTS, RPT), jnp.float32),
        pltpu.SemaphoreType.DMA,
    ],
)
def _sc_gather(flat_hbm, tgt_hbm, noise_hbm, out_hbm, tgt_v, nz_v, idx_v, val_v, sem):
    wid = lax.axis_index("s") * NC + lax.axis_index("c")
    base = wid * RPT
    pltpu.sync_copy(tgt_hbm.at[pl.ds(base, RPT)], tgt_v)
    pltpu.sync_copy(noise_hbm.at[pl.ds(base * NNEG, RPT * NNEG)], nz_v)
    lanes = lax.iota(jnp.int32, L)
    for i in range(RPT // L):
        rows = base + i * L + lanes
        idx_v[0, pl.ds(i * L, L)] = rows * V + tgt_v[pl.ds(i * L, L)]
    for j in range(NNEG):
        for i in range(RPT // L):
            q = j * RPT + i * L + lanes
            rows = base + ((q * 6554) >> 15)
            idx_v[1 + j, pl.ds(i * L, L)] = rows * V + nz_v[pl.ds(j * RPT + i * L, L)]
    copies = [
        pltpu.async_copy(flat_hbm.at[idx_v.at[j]], val_v.at[j], sem)
        for j in range(SLOTS)
    ]
    for c in copies:
        c.wait()
    pltpu.sync_copy(val_v, out_hbm.at[wid])


def _loss_body(x_ref, o_ref):
    x = x_ref[...]
    r = lax.broadcasted_iota(jnp.int32, x.shape, 0)
    y = jnp.where(r % SLOTS == 0, x, -x)
    o_ref[...] = (-jnp.sum(jnp.log(jax.nn.sigmoid(y))) / B).reshape(1, 1)


_tc_loss = pl.pallas_call(
    _loss_body,
    out_shape=jax.ShapeDtypeStruct((1, 1), jnp.float32),
)


def kernel(output, target):
    noise_key = jax.random.fold_in(jax.random.key(0), 1234)
    noise = jax.random.randint(noise_key, (B, NNEG), 0, V)
    vals = _sc_gather(
        output.reshape(-1),
        target.astype(jnp.int32),
        noise.reshape(-1).astype(jnp.int32),
    )
    return _tc_loss(vals.reshape(NW * SLOTS, RPT))[0, 0]


# speedup vs baseline: 2.3012x; 2.3012x over previous
import functools

import jax
import jax.numpy as jnp
import numpy as np
from jax import lax
from jax.experimental import pallas as pl
from jax.experimental.pallas import tpu as pltpu
from jax.experimental.pallas import tpu_sc as plsc

V = 100000
NNEG = 5
B = 4096
NC, NS, L = 2, 16, 16
NW = NC * NS
RPT = B // NW
SLOTS = 1 + NNEG
EPT = SLOTS * RPT
CW = 128
GW = 16
WIN = 64
NWIN = EPT // WIN

_mesh = plsc.VectorSubcoreMesh(core_axis_name="c", subcore_axis_name="s")

_E = np.arange(EPT)
_ROWS = (np.arange(NW)[:, None] * RPT
         + np.where(_E < RPT, _E, (_E - RPT) // NNEG)[None, :])
_ROWS = _ROWS.astype(np.int32).reshape(NW, EPT, 1)


@functools.partial(
    pl.kernel,
    mesh=_mesh,
    out_type=(
        jax.ShapeDtypeStruct((NW, EPT * GW), jnp.float32),
        jax.ShapeDtypeStruct((NW, EPT), jnp.int32),
    ),
    scratch_types=[
        pltpu.VMEM((RPT,), jnp.int32),
        pltpu.VMEM((NNEG * RPT,), jnp.int32),
        pltpu.VMEM((EPT, 1), jnp.int32),
        pltpu.VMEM((EPT,), jnp.int32),
        pltpu.VMEM((EPT,), jnp.int32),
        pltpu.VMEM((EPT,), jnp.int32),
        pltpu.VMEM((2, WIN, CW), jnp.float32),
        pltpu.VMEM_SHARED((NS, WIN * GW), jnp.float32),
        pltpu.SemaphoreType.DMA,
        pltpu.SemaphoreType.DMA,
        pltpu.SemaphoreType.DMA,
    ],
)
def _sc_gather(tab_hbm, tgt_hbm, noise_hbm, rows_hbm, val_hbm, lane_hbm,
               tgt_v, nz_v, row_v, col_v, gb_v, lane_v, stage_v, out_sh,
               sem0, sem1, sem2):
    sid = lax.axis_index("s")
    wid = sid * NC + lax.axis_index("c")
    base = wid * RPT
    pltpu.sync_copy(tgt_hbm.at[pl.ds(base, RPT)], tgt_v)
    pltpu.sync_copy(noise_hbm.at[pl.ds(base * NNEG, RPT * NNEG)], nz_v)
    pltpu.sync_copy(rows_hbm.at[wid], row_v)
    for g in range(EPT // L):
        if g < RPT // L:
            c = tgt_v[pl.ds(g * L, L)]
        else:
            c = nz_v[pl.ds(g * L - RPT, L)]
        col_v[pl.ds(g * L, L)] = c & -CW
        gb_v[pl.ds(g * L, L)] = c & (CW - GW)
        lane_v[pl.ds(g * L, L)] = c & (GW - 1)
    pltpu.sync_copy(lane_v, lane_hbm.at[wid])

    sems = (sem0, sem1)

    def _issue(w, buf, sem):
        def body(g, _):
            off = w * WIN + g * L
            cc = col_v[pl.ds(off, L)]
            for k in range(L):
                pltpu.async_copy(
                    tab_hbm.at[
                        row_v.at[off + k],
                        pl.ds(pl.multiple_of(cc[k], CW), CW),
                    ],
                    stage_v.at[buf, pl.ds(g * L + k, 1)],
                    sem,
                )
            return _
        lax.fori_loop(0, WIN // L, body, 0)

    def _drain_fetch(sem):
        pltpu.make_async_copy(
            tab_hbm.at[pl.ds(0, WIN), pl.ds(0, CW)],
            stage_v.at[0],
            sem,
        ).wait()

    _issue(0, 0, sems[0])
    for w in range(NWIN):
        if w + 1 < NWIN:
            _issue(w + 1, (w + 1) % 2, sems[(w + 1) % 2])
        _drain_fetch(sems[w % 2])
        buf = w % 2

        def _compact(g, _, buf=buf):
            off = w * WIN + g * L
            gg = gb_v[pl.ds(off, L)]
            for k in range(L):
                pltpu.async_copy(
                    stage_v.at[
                        buf,
                        g * L + k,
                        pl.ds(pl.multiple_of(gg[k], GW), GW),
                    ],
                    out_sh.at[sid, pl.ds((g * L + k) * GW, GW)],
                    sem2,
                )
            return _

        lax.fori_loop(0, WIN // L, _compact, 0)
        pltpu.make_async_copy(
            val_hbm.at[wid, pl.ds(0, WIN * GW)], out_sh.at[sid], sem2
        ).wait()
        pltpu.sync_copy(
            out_sh.at[sid], val_hbm.at[wid, pl.ds(w * WIN * GW, WIN * GW)]
        )


def _loss_body(x_ref, lane_ref, o_ref):
    x = x_ref[...]
    ln = lane_ref[...]
    sel = lax.broadcasted_iota(jnp.int32, x.shape, 2) == ln[:, :, None]
    y = jnp.sum(jnp.where(sel, x, 0.0), axis=2)
    r = lax.broadcasted_iota(jnp.int32, y.shape, 0)
    y = jnp.where(r % SLOTS == 0, y, -y)
    o_ref[...] = (-jnp.sum(jnp.log(jax.nn.sigmoid(y))) / B).reshape(1, 1)


_tc_loss = pl.pallas_call(
    _loss_body,
    out_shape=jax.ShapeDtypeStruct((1, 1), jnp.float32),
)


def kernel(output, target):
    noise_key = jax.random.fold_in(jax.random.key(0), 1234)
    noise = jax.random.randint(noise_key, (B, NNEG), 0, V)
    vals, lanes = _sc_gather(
        output,
        target.astype(jnp.int32),
        noise.reshape(-1).astype(jnp.int32),
        jnp.asarray(_ROWS),
    )
    return _tc_loss(
        vals.reshape(NW * EPT // CW, CW, GW),
        lanes.reshape(NW * EPT // CW, CW),
    )[0, 0]


# speedup vs baseline: 54.3768x; 23.6302x over previous
import functools

import jax
import jax.numpy as jnp
import numpy as np
from jax import lax
from jax.experimental import pallas as pl
from jax.experimental.pallas import tpu as pltpu
from jax.experimental.pallas import tpu_sc as plsc

V = 100000
NNEG = 5
B = 4096
NC, NS, L = 2, 16, 16
NW = NC * NS
RPT = B // NW
SLOTS = 1 + NNEG
EPT = SLOTS * RPT
CW = 128
GW = 8
WIN = 128

_mesh = plsc.VectorSubcoreMesh(core_axis_name="c", subcore_axis_name="s")

_E = np.arange(EPT)
_LANE = np.where(_E < RPT, _E, (_E - RPT) // NNEG).astype(int)


@functools.partial(
    pl.kernel,
    mesh=_mesh,
    out_type=jax.ShapeDtypeStruct((NW, EPT * GW), jnp.float32),
    scratch_types=[
        pltpu.VMEM((RPT,), jnp.int32),
        pltpu.VMEM((NNEG * RPT,), jnp.int32),
        pltpu.VMEM((2, WIN, CW), jnp.float32),
        pltpu.VMEM_SHARED((NS, EPT * GW), jnp.float32),
        pltpu.SemaphoreType.DMA,
        pltpu.SemaphoreType.DMA,
        pltpu.SemaphoreType.DMA,
    ],
)
def _sc_gather(tabt_hbm, tgt_hbm, noise_hbm, val_hbm,
               tgt_v, nz_v, stage_v, out_sh, sem0, sem1, sem2):
    sid = lax.axis_index("s")
    wid = sid * NC + lax.axis_index("c")
    base = wid * RPT
    pltpu.sync_copy(tgt_hbm.at[pl.ds(base, RPT)], tgt_v)
    pltpu.sync_copy(noise_hbm.at[pl.ds(base * NNEG, RPT * NNEG)], nz_v)
    sems = (sem0, sem1)

    def _fetch(w):
        idx = tgt_v if w == 0 else nz_v.at[pl.ds((w - 1) * WIN, WIN)]
        return pltpu.async_copy(
            tabt_hbm.at[idx, pl.ds(pl.multiple_of(base, CW), CW)],
            stage_v.at[w % 2],
            sems[w % 2],
        )

    fetches = [None] * SLOTS
    fetches[0] = _fetch(0)
    for w in range(SLOTS):
        if w + 1 < SLOTS:
            fetches[w + 1] = _fetch(w + 1)
        fetches[w].wait()
        for k in range(WIN):
            e = w * WIN + k
            pltpu.async_copy(
                stage_v.at[w % 2, k, pl.ds(int(_LANE[e]) & -GW, GW)],
                out_sh.at[sid, pl.ds(e * GW, GW)],
                sem2,
            )
        pltpu.make_async_copy(
            val_hbm.at[wid, pl.ds(0, WIN * GW)],
            out_sh.at[sid, pl.ds(w * WIN * GW, WIN * GW)],
            sem2,
        ).wait()
    pltpu.sync_copy(out_sh.at[sid], val_hbm.at[wid])


def _loss_body(x_ref, o_ref):
    x = x_ref[...]
    rows = lax.broadcasted_iota(jnp.int32, x.shape[:2], 0)
    cols = lax.broadcasted_iota(jnp.int32, x.shape[:2], 1)
    slot = rows % SLOTS
    p = (slot - 1) * RPT + cols
    lane = jnp.where(slot == 0, cols, (p * 6554) >> 15) & (GW - 1)
    sel = lax.broadcasted_iota(jnp.int32, x.shape, 2) == lane[:, :, None]
    y = jnp.sum(jnp.where(sel, x, 0.0), axis=2)
    y = jnp.where(slot == 0, y, -y)
    o_ref[...] = (-jnp.sum(jnp.log(jax.nn.sigmoid(y))) / B).reshape(1, 1)


_tc_loss = pl.pallas_call(
    _loss_body,
    out_shape=jax.ShapeDtypeStruct((1, 1), jnp.float32),
)


def kernel(output, target):
    noise_key = jax.random.fold_in(jax.random.key(0), 1234)
    noise = jax.random.randint(noise_key, (B, NNEG), 0, V)
    vals = _sc_gather(
        output.T,
        target.astype(jnp.int32),
        noise.reshape(-1).astype(jnp.int32),
    )
    return _tc_loss(vals.reshape(NW * SLOTS, RPT, GW))[0, 0]
